# baseline (speedup 1.0000x reference)
import functools

import jax
import jax.numpy as jnp
from jax import lax
from jax.experimental import pallas as pl
from jax.experimental.pallas import tpu as pltpu
from jax.experimental.pallas import tpu_sc as plsc

N = 10000
E = 320000
F = 128
D = 128

NC = 2
NS = 16
NW = NC * NS

EPT = E // NW
K = 80
NCHUNK = EPT // K
KA = 80
NCA = EPT // KA

NPAD = 10240
DSL = NPAD // NS
DW = 16
NACC = 10240
TPTP = NACC // NS

_EPS = 1e-5


@functools.cache
def _sc_kernels():
    mesh = plsc.VectorSubcoreMesh(core_axis_name="c", subcore_axis_name="s")

    @functools.partial(
        pl.kernel,
        mesh=mesh,
        compiler_params=pltpu.CompilerParams(needs_layout_passes=False),
        out_type=jax.ShapeDtypeStruct((NW, 1, NPAD), jnp.float32),
        scratch_types=[
            pltpu.VMEM((EPT,), jnp.int32),
            pltpu.VMEM((NPAD,), jnp.float32),
        ],
    )
    def sc_degree(col_hbm, zeros_hbm, out_hbm, colm, dacc):
        c = lax.axis_index("c")
        s = lax.axis_index("s")
        wid = s * NC + c
        pltpu.sync_copy(col_hbm.at[pl.ds(wid * EPT, EPT)], colm)
        pltpu.sync_copy(zeros_hbm, dacc)
        ones16 = jnp.ones((16,), jnp.float32)

        def chunk(t, carry):
            idx = colm[pl.ds(t * 16, 16)]
            plsc.addupdate_scatter(dacc, [idx], ones16)
            return carry

        lax.fori_loop(0, EPT // 16, chunk, 0)
        pltpu.sync_copy(dacc, out_hbm.at[wid, 0])

    @functools.partial(
        pl.kernel,
        mesh=mesh,
        out_type=jax.ShapeDtypeStruct((NC, NACC, D), jnp.float32),
        scratch_types=[
            [pltpu.VMEM((KA,), jnp.int32) for _ in range(4)],
            [pltpu.VMEM((KA,), jnp.int32) for _ in range(4)],
            [pltpu.VMEM((KA, D), jnp.float32) for _ in range(4)],
            pltpu.VMEM_SHARED((NACC, D), jnp.float32),
            [pltpu.SemaphoreType.DMA for _ in range(4)],
            [pltpu.SemaphoreType.DMA for _ in range(4)],
            [pltpu.SemaphoreType.DMA for _ in range(4)],
            [pltpu.SemaphoreType.DMA for _ in range(4)],
        ],
    )
    def sc_aggregate(h_hbm, row_hbm, col_hbm, zeros_hbm, out_hbm,
                     r, c, buf, acc, g, sc, ir, ic):
        cc_ = lax.axis_index("c")
        ss_ = lax.axis_index("s")
        wid = ss_ * NC + cc_
        pltpu.sync_copy(zeros_hbm, acc.at[pl.ds(ss_ * TPTP, TPTP)])

        base = wid * EPT

        def ldrow(i, b):
            pltpu.async_copy(row_hbm.at[pl.ds(base + i * KA, KA)], r[b], ir[b])

        def ldcol(i, b):
            pltpu.async_copy(col_hbm.at[pl.ds(base + i * KA, KA)], c[b], ic[b])

        def irwait(b):
            pltpu.make_async_copy(row_hbm.at[pl.ds(base, KA)], r[b], ir[b]).wait()

        def icwait(b):
            pltpu.make_async_copy(col_hbm.at[pl.ds(base, KA)], c[b], ic[b]).wait()

        def gather(b):
            pltpu.async_copy(h_hbm.at[r[b]], buf[b], g[b])

        def gwait(b):
            pltpu.make_async_copy(h_hbm.at[r[0]], buf[b], g[b]).wait()

        def scat(b):
            pltpu.async_copy(buf[b], acc.at[c[b]], sc[b], add=True)

        def swait(b):
            pltpu.make_async_copy(buf[b], acc.at[c[0]], sc[b]).wait()

        for b in range(4):
            ldrow(b, b)
        ldcol(0, 0)
        ldcol(1, 1)
        plsc.subcore_barrier()
        irwait(0)
        gather(0)
        irwait(1)
        gather(1)

        def step(i, b):
            b2 = (b + 2) % 4
            gwait(b)
            @pl.when(i + 4 < NCA)
            def _ldr():
                ldrow(i + 4, b)
            icwait(b)
            scat(b)
            @pl.when(i + 2 < NCA)
            def _nxt():
                @pl.when(i >= 2)
                def _sw():
                    swait(b2)
                ldcol(i + 2, b2)
                irwait(b2)
                gather(b2)

        def group(j, carry):
            i0 = 4 * j
            for b in range(4):
                step(i0 + b, b)
            return carry

        lax.fori_loop(0, NCA // 4, group, 0)
        for i in range(NCA - (NCA % 4), NCA):
            step(i, i % 4)
        for b in range(4):
            swait(b)
        plsc.subcore_barrier()
        pltpu.sync_copy(acc.at[pl.ds(ss_ * TPTP, TPTP)],
                        out_hbm.at[cc_, pl.ds(ss_ * TPTP, TPTP)])

    return sc_degree, sc_aggregate


def _tc_dinv_body(deg_ref, out_ref):
    deg = jnp.sum(deg_ref[:, 0, :N], axis=0, keepdims=True) + 1.0
    out_ref[...] = lax.rsqrt(deg).reshape(N, 1)


def _tc_first_body(x_ref, w_ref, dinv_ref, out_ref):
    h = jnp.dot(x_ref[...], w_ref[...], preferred_element_type=jnp.float32)
    out_ref[...] = h * dinv_ref[...]


def _tc_mid_body(s_ref, hp_ref, dinv_ref, b_ref, g_ref, be_ref, w_ref, out_ref):
    dinv = dinv_ref[...]
    conv = (s_ref[0, :N] + s_ref[1, :N] + hp_ref[...]) * dinv + b_ref[...]
    m = jnp.mean(conv, axis=0, keepdims=True)
    cc = conv - m
    v = jnp.mean(cc * cc, axis=0, keepdims=True)
    y = cc * lax.rsqrt(v + _EPS) * g_ref[...] + be_ref[...]
    y = jnp.maximum(y, 0.0)
    out_ref[...] = jnp.dot(y, w_ref[...],
                           preferred_element_type=jnp.float32) * dinv


def _tc_last_body(s_ref, hp_ref, dinv_ref, b_ref, g_ref, be_ref, out_ref):
    conv = (s_ref[0, :N] + s_ref[1, :N] + hp_ref[...]) * dinv_ref[...] + b_ref[...]
    m = jnp.mean(conv, axis=0, keepdims=True)
    cc = conv - m
    v = jnp.mean(cc * cc, axis=0, keepdims=True)
    out_ref[...] = cc * lax.rsqrt(v + _EPS) * g_ref[...] + be_ref[...]


_nd_f32 = jax.ShapeDtypeStruct((N, D), jnp.float32)

_tc_dinv = pl.pallas_call(
    _tc_dinv_body, out_shape=jax.ShapeDtypeStruct((N, 1), jnp.float32))
_tc_first = pl.pallas_call(_tc_first_body, out_shape=_nd_f32)
_tc_mid = pl.pallas_call(_tc_mid_body, out_shape=_nd_f32)
_tc_last = pl.pallas_call(_tc_last_body, out_shape=_nd_f32)


def kernel(x, edge_index, W1, b1, g1, be1, W2, b2, g2, be2, W3, b3, g3, be3):
    sc_degree, sc_aggregate = _sc_kernels()
    row_mat = edge_index[0]
    col_mat = edge_index[1]

    zeros_deg = jnp.zeros((NPAD,), jnp.float32)
    zeros_acc = jnp.zeros((TPTP, D), jnp.float32)

    deg_part = sc_degree(col_mat, zeros_deg)
    dinv = _tc_dinv(deg_part)

    b1r, g1r, be1r = b1.reshape(1, D), g1.reshape(1, D), be1.reshape(1, D)
    b2r, g2r, be2r = b2.reshape(1, D), g2.reshape(1, D), be2.reshape(1, D)
    b3r, g3r, be3r = b3.reshape(1, D), g3.reshape(1, D), be3.reshape(1, D)

    hp = _tc_first(x, W1, dinv)
    S = sc_aggregate(hp, row_mat, col_mat, zeros_acc)
    hp = _tc_mid(S, hp, dinv, b1r, g1r, be1r, W2)
    S = sc_aggregate(hp, row_mat, col_mat, zeros_acc)
    hp = _tc_mid(S, hp, dinv, b2r, g2r, be2r, W3)
    S = sc_aggregate(hp, row_mat, col_mat, zeros_acc)
    return _tc_last(S, hp, dinv, b3r, g3r, be3r)


# speedup vs baseline: 1.0978x; 1.0978x over previous
import functools

import jax
import jax.numpy as jnp
from jax import lax
from jax.experimental import pallas as pl
from jax.experimental.pallas import tpu as pltpu
from jax.experimental.pallas import tpu_sc as plsc

N = 10000
E = 320000
F = 128
D = 128

NC = 2
NS = 16
NW = NC * NS

EPT = E // NW
K = 80
NCHUNK = EPT // K
KA = 80
NCA = EPT // KA

NB = 4
G = 3
NPAD = 10240
DSL = NPAD // NS
DW = 16
NACC = 10240
TPTP = NACC // NS

_EPS = 1e-5


@functools.cache
def _sc_kernels():
    mesh = plsc.VectorSubcoreMesh(core_axis_name="c", subcore_axis_name="s")

    @functools.partial(
        pl.kernel,
        mesh=mesh,
        compiler_params=pltpu.CompilerParams(needs_layout_passes=False),
        out_type=jax.ShapeDtypeStruct((NW, 1, NPAD), jnp.float32),
        scratch_types=[
            pltpu.VMEM((EPT,), jnp.int32),
            pltpu.VMEM((NPAD,), jnp.float32),
        ],
    )
    def sc_degree(col_hbm, zeros_hbm, out_hbm, colm, dacc):
        c = lax.axis_index("c")
        s = lax.axis_index("s")
        wid = s * NC + c
        pltpu.sync_copy(col_hbm.at[pl.ds(wid * EPT, EPT)], colm)
        pltpu.sync_copy(zeros_hbm, dacc)
        ones16 = jnp.ones((16,), jnp.float32)

        def chunk(t, carry):
            idx = colm[pl.ds(t * 16, 16)]
            plsc.addupdate_scatter(dacc, [idx], ones16)
            return carry

        lax.fori_loop(0, EPT // 16, chunk, 0)
        pltpu.sync_copy(dacc, out_hbm.at[wid, 0])

    @functools.partial(
        pl.kernel,
        mesh=mesh,
        out_type=jax.ShapeDtypeStruct((NC, NACC, D), jnp.float32),
        scratch_types=[
            [pltpu.VMEM((KA,), jnp.int32) for _ in range(NB)],
            [pltpu.VMEM((KA,), jnp.int32) for _ in range(NB)],
            [pltpu.VMEM((KA, D), jnp.float32) for _ in range(NB)],
            pltpu.VMEM_SHARED((NACC, D), jnp.float32),
            [pltpu.SemaphoreType.DMA for _ in range(NB)],
            [pltpu.SemaphoreType.DMA for _ in range(NB)],
            [pltpu.SemaphoreType.DMA for _ in range(NB)],
            [pltpu.SemaphoreType.DMA for _ in range(NB)],
        ],
    )
    def sc_aggregate(h_hbm, row_hbm, col_hbm, zeros_hbm, out_hbm,
                     r, c, buf, acc, g, sc, ir, ic):
        cc_ = lax.axis_index("c")
        ss_ = lax.axis_index("s")
        wid = ss_ * NC + cc_
        pltpu.sync_copy(zeros_hbm, acc.at[pl.ds(ss_ * TPTP, TPTP)])

        base = wid * EPT

        def ldrow(i, b):
            pltpu.async_copy(row_hbm.at[pl.ds(base + i * KA, KA)], r[b], ir[b])

        def ldcol(i, b):
            pltpu.async_copy(col_hbm.at[pl.ds(base + i * KA, KA)], c[b], ic[b])

        def irwait(b):
            pltpu.make_async_copy(row_hbm.at[pl.ds(base, KA)], r[b], ir[b]).wait()

        def icwait(b):
            pltpu.make_async_copy(col_hbm.at[pl.ds(base, KA)], c[b], ic[b]).wait()

        def gather(b):
            pltpu.async_copy(h_hbm.at[r[b]], buf[b], g[b])

        def gwait(b):
            pltpu.make_async_copy(h_hbm.at[r[0]], buf[b], g[b]).wait()

        def scat(b):
            pltpu.async_copy(buf[b], acc.at[c[b]], sc[b], add=True)

        def swait(b):
            pltpu.make_async_copy(buf[b], acc.at[c[0]], sc[b]).wait()

        for b in range(NB):
            ldrow(b, b)
        for b in range(G):
            ldcol(b, b)
        plsc.subcore_barrier()
        for b in range(G):
            irwait(b)
            gather(b)

        def step(i, b):
            b3 = (b + G) % NB
            gwait(b)
            @pl.when(i + NB < NCA)
            def _ldr():
                ldrow(i + NB, b)
            icwait(b)
            scat(b)
            @pl.when(i + G < NCA)
            def _nxt():
                @pl.when(i + G - NB >= 0)
                def _sw():
                    swait(b3)
                ldcol(i + G, b3)
                irwait(b3)
                gather(b3)

        def group(j, carry):
            i0 = NB * j
            for b in range(NB):
                step(i0 + b, b)
            return carry

        lax.fori_loop(0, NCA // NB, group, 0)
        for i in range(NCA - (NCA % NB), NCA):
            step(i, i % NB)
        for i in range(NCA - NB, NCA):
            swait(i % NB)
        plsc.subcore_barrier()
        pltpu.sync_copy(acc.at[pl.ds(ss_ * TPTP, TPTP)],
                        out_hbm.at[cc_, pl.ds(ss_ * TPTP, TPTP)])

    return sc_degree, sc_aggregate


def _tc_dinv_body(deg_ref, out_ref):
    deg = jnp.sum(deg_ref[:, 0, :N], axis=0, keepdims=True) + 1.0
    out_ref[...] = lax.rsqrt(deg).reshape(N, 1)


def _tc_first_body(x_ref, w_ref, dinv_ref, out_ref):
    h = jnp.dot(x_ref[...], w_ref[...], preferred_element_type=jnp.float32)
    out_ref[...] = h * dinv_ref[...]


def _tc_mid_body(s_ref, hp_ref, dinv_ref, b_ref, g_ref, be_ref, w_ref, out_ref):
    dinv = dinv_ref[...]
    conv = (s_ref[0, :N] + s_ref[1, :N] + hp_ref[...]) * dinv + b_ref[...]
    m = jnp.mean(conv, axis=0, keepdims=True)
    cc = conv - m
    v = jnp.mean(cc * cc, axis=0, keepdims=True)
    y = cc * lax.rsqrt(v + _EPS) * g_ref[...] + be_ref[...]
    y = jnp.maximum(y, 0.0)
    out_ref[...] = jnp.dot(y, w_ref[...],
                           preferred_element_type=jnp.float32) * dinv


def _tc_last_body(s_ref, hp_ref, dinv_ref, b_ref, g_ref, be_ref, out_ref):
    conv = (s_ref[0, :N] + s_ref[1, :N] + hp_ref[...]) * dinv_ref[...] + b_ref[...]
    m = jnp.mean(conv, axis=0, keepdims=True)
    cc = conv - m
    v = jnp.mean(cc * cc, axis=0, keepdims=True)
    out_ref[...] = cc * lax.rsqrt(v + _EPS) * g_ref[...] + be_ref[...]


_nd_f32 = jax.ShapeDtypeStruct((N, D), jnp.float32)

_tc_dinv = pl.pallas_call(
    _tc_dinv_body, out_shape=jax.ShapeDtypeStruct((N, 1), jnp.float32))
_tc_first = pl.pallas_call(_tc_first_body, out_shape=_nd_f32)
_tc_mid = pl.pallas_call(_tc_mid_body, out_shape=_nd_f32)
_tc_last = pl.pallas_call(_tc_last_body, out_shape=_nd_f32)


def kernel(x, edge_index, W1, b1, g1, be1, W2, b2, g2, be2, W3, b3, g3, be3):
    sc_degree, sc_aggregate = _sc_kernels()
    row_mat = edge_index[0]
    col_mat = edge_index[1]

    zeros_deg = jnp.zeros((NPAD,), jnp.float32)
    zeros_acc = jnp.zeros((TPTP, D), jnp.float32)

    deg_part = sc_degree(col_mat, zeros_deg)
    dinv = _tc_dinv(deg_part)

    b1r, g1r, be1r = b1.reshape(1, D), g1.reshape(1, D), be1.reshape(1, D)
    b2r, g2r, be2r = b2.reshape(1, D), g2.reshape(1, D), be2.reshape(1, D)
    b3r, g3r, be3r = b3.reshape(1, D), g3.reshape(1, D), be3.reshape(1, D)

    hp = _tc_first(x, W1, dinv)
    S = sc_aggregate(hp, row_mat, col_mat, zeros_acc)
    hp = _tc_mid(S, hp, dinv, b1r, g1r, be1r, W2)
    S = sc_aggregate(hp, row_mat, col_mat, zeros_acc)
    hp = _tc_mid(S, hp, dinv, b2r, g2r, be2r, W3)
    S = sc_aggregate(hp, row_mat, col_mat, zeros_acc)
    return _tc_last(S, hp, dinv, b3r, g3r, be3r)


# speedup vs baseline: 1.1298x; 1.0291x over previous
import functools

import jax
import jax.numpy as jnp
from jax import lax
from jax.experimental import pallas as pl
from jax.experimental.pallas import tpu as pltpu
from jax.experimental.pallas import tpu_sc as plsc

N = 10000
E = 320000
F = 128
D = 128

NC = 2
NS = 16
NW = NC * NS

EPT = E // NW
K = 80
NCHUNK = EPT // K
KA = 40
NCA = EPT // KA

NB = 6
G = 4
NPAD = 10240
DSL = NPAD // NS
DW = 16
NACC = 10240
TPTP = NACC // NS

_EPS = 1e-5


@functools.cache
def _sc_kernels():
    mesh = plsc.VectorSubcoreMesh(core_axis_name="c", subcore_axis_name="s")

    @functools.partial(
        pl.kernel,
        mesh=mesh,
        compiler_params=pltpu.CompilerParams(needs_layout_passes=False),
        out_type=jax.ShapeDtypeStruct((NW, 1, NPAD), jnp.float32),
        scratch_types=[
            pltpu.VMEM((EPT,), jnp.int32),
            pltpu.VMEM((NPAD,), jnp.float32),
        ],
    )
    def sc_degree(col_hbm, zeros_hbm, out_hbm, colm, dacc):
        c = lax.axis_index("c")
        s = lax.axis_index("s")
        wid = s * NC + c
        pltpu.sync_copy(col_hbm.at[pl.ds(wid * EPT, EPT)], colm)
        pltpu.sync_copy(zeros_hbm, dacc)
        ones16 = jnp.ones((16,), jnp.float32)

        def chunk(t, carry):
            idx = colm[pl.ds(t * 16, 16)]
            plsc.addupdate_scatter(dacc, [idx], ones16)
            return carry

        lax.fori_loop(0, EPT // 16, chunk, 0)
        pltpu.sync_copy(dacc, out_hbm.at[wid, 0])

    @functools.partial(
        pl.kernel,
        mesh=mesh,
        out_type=jax.ShapeDtypeStruct((NC, NACC, D), jnp.float32),
        scratch_types=[
            [pltpu.VMEM((KA,), jnp.int32) for _ in range(NB)],
            [pltpu.VMEM((KA,), jnp.int32) for _ in range(NB)],
            [pltpu.VMEM((KA, D), jnp.float32) for _ in range(NB)],
            pltpu.VMEM_SHARED((NACC, D), jnp.float32),
            [pltpu.SemaphoreType.DMA for _ in range(NB)],
            [pltpu.SemaphoreType.DMA for _ in range(NB)],
            [pltpu.SemaphoreType.DMA for _ in range(NB)],
            [pltpu.SemaphoreType.DMA for _ in range(NB)],
        ],
    )
    def sc_aggregate(h_hbm, row_hbm, col_hbm, zeros_hbm, out_hbm,
                     r, c, buf, acc, g, sc, ir, ic):
        cc_ = lax.axis_index("c")
        ss_ = lax.axis_index("s")
        wid = ss_ * NC + cc_
        pltpu.sync_copy(zeros_hbm, acc.at[pl.ds(ss_ * TPTP, TPTP)])

        base = wid * EPT

        def ldrow(i, b):
            pltpu.async_copy(row_hbm.at[pl.ds(base + i * KA, KA)], r[b], ir[b])

        def ldcol(i, b):
            pltpu.async_copy(col_hbm.at[pl.ds(base + i * KA, KA)], c[b], ic[b])

        def irwait(b):
            pltpu.make_async_copy(row_hbm.at[pl.ds(base, KA)], r[b], ir[b]).wait()

        def icwait(b):
            pltpu.make_async_copy(col_hbm.at[pl.ds(base, KA)], c[b], ic[b]).wait()

        def gather(b):
            pltpu.async_copy(h_hbm.at[r[b]], buf[b], g[b])

        def gwait(b):
            pltpu.make_async_copy(h_hbm.at[r[0]], buf[b], g[b]).wait()

        def scat(b):
            pltpu.async_copy(buf[b], acc.at[c[b]], sc[b], add=True)

        def swait(b):
            pltpu.make_async_copy(buf[b], acc.at[c[0]], sc[b]).wait()

        for b in range(NB):
            ldrow(b, b)
        for b in range(G):
            ldcol(b, b)
        plsc.subcore_barrier()
        for b in range(G):
            irwait(b)
            gather(b)

        def step(i, b):
            b3 = (b + G) % NB
            gwait(b)
            @pl.when(i + NB < NCA)
            def _ldr():
                ldrow(i + NB, b)
            icwait(b)
            scat(b)
            @pl.when(i + G < NCA)
            def _nxt():
                @pl.when(i + G - NB >= 0)
                def _sw():
                    swait(b3)
                ldcol(i + G, b3)
                irwait(b3)
                gather(b3)

        def group(j, carry):
            i0 = NB * j
            for b in range(NB):
                step(i0 + b, b)
            return carry

        lax.fori_loop(0, NCA // NB, group, 0)
        for i in range(NCA - (NCA % NB), NCA):
            step(i, i % NB)
        for i in range(NCA - NB, NCA):
            swait(i % NB)
        plsc.subcore_barrier()
        pltpu.sync_copy(acc.at[pl.ds(ss_ * TPTP, TPTP)],
                        out_hbm.at[cc_, pl.ds(ss_ * TPTP, TPTP)])

    return sc_degree, sc_aggregate


def _tc_dinv_body(deg_ref, out_ref):
    deg = jnp.sum(deg_ref[:, 0, :N], axis=0, keepdims=True) + 1.0
    out_ref[...] = lax.rsqrt(deg).reshape(N, 1)


def _tc_first_body(x_ref, w_ref, dinv_ref, out_ref):
    h = jnp.dot(x_ref[...], w_ref[...], preferred_element_type=jnp.float32)
    out_ref[...] = h * dinv_ref[...]


def _tc_mid_body(s_ref, hp_ref, dinv_ref, b_ref, g_ref, be_ref, w_ref, out_ref):
    dinv = dinv_ref[...]
    conv = (s_ref[0, :N] + s_ref[1, :N] + hp_ref[...]) * dinv + b_ref[...]
    m = jnp.mean(conv, axis=0, keepdims=True)
    cc = conv - m
    v = jnp.mean(cc * cc, axis=0, keepdims=True)
    y = cc * lax.rsqrt(v + _EPS) * g_ref[...] + be_ref[...]
    y = jnp.maximum(y, 0.0)
    out_ref[...] = jnp.dot(y, w_ref[...],
                           preferred_element_type=jnp.float32) * dinv


def _tc_last_body(s_ref, hp_ref, dinv_ref, b_ref, g_ref, be_ref, out_ref):
    conv = (s_ref[0, :N] + s_ref[1, :N] + hp_ref[...]) * dinv_ref[...] + b_ref[...]
    m = jnp.mean(conv, axis=0, keepdims=True)
    cc = conv - m
    v = jnp.mean(cc * cc, axis=0, keepdims=True)
    out_ref[...] = cc * lax.rsqrt(v + _EPS) * g_ref[...] + be_ref[...]


_nd_f32 = jax.ShapeDtypeStruct((N, D), jnp.float32)

_tc_dinv = pl.pallas_call(
    _tc_dinv_body, out_shape=jax.ShapeDtypeStruct((N, 1), jnp.float32))
_tc_first = pl.pallas_call(_tc_first_body, out_shape=_nd_f32)
_tc_mid = pl.pallas_call(_tc_mid_body, out_shape=_nd_f32)
_tc_last = pl.pallas_call(_tc_last_body, out_shape=_nd_f32)


def kernel(x, edge_index, W1, b1, g1, be1, W2, b2, g2, be2, W3, b3, g3, be3):
    sc_degree, sc_aggregate = _sc_kernels()
    row_mat = edge_index[0]
    col_mat = edge_index[1]

    zeros_deg = jnp.zeros((NPAD,), jnp.float32)
    zeros_acc = jnp.zeros((TPTP, D), jnp.float32)

    deg_part = sc_degree(col_mat, zeros_deg)
    dinv = _tc_dinv(deg_part)

    b1r, g1r, be1r = b1.reshape(1, D), g1.reshape(1, D), be1.reshape(1, D)
    b2r, g2r, be2r = b2.reshape(1, D), g2.reshape(1, D), be2.reshape(1, D)
    b3r, g3r, be3r = b3.reshape(1, D), g3.reshape(1, D), be3.reshape(1, D)

    hp = _tc_first(x, W1, dinv)
    S = sc_aggregate(hp, row_mat, col_mat, zeros_acc)
    hp = _tc_mid(S, hp, dinv, b1r, g1r, be1r, W2)
    S = sc_aggregate(hp, row_mat, col_mat, zeros_acc)
    hp = _tc_mid(S, hp, dinv, b2r, g2r, be2r, W3)
    S = sc_aggregate(hp, row_mat, col_mat, zeros_acc)
    return _tc_last(S, hp, dinv, b3r, g3r, be3r)


# speedup vs baseline: 1.1442x; 1.0128x over previous
import functools

import jax
import jax.numpy as jnp
from jax import lax
from jax.experimental import pallas as pl
from jax.experimental.pallas import tpu as pltpu
from jax.experimental.pallas import tpu_sc as plsc

N = 10000
E = 320000
F = 128
D = 128

NC = 2
NS = 16
NW = NC * NS

EPT = E // NW
K = 80
NCHUNK = EPT // K
KA = 40
NCA = EPT // KA

NB = 6
G = 5
NPAD = 10240
DSL = NPAD // NS
DW = 16
NACC = 10240
TPTP = NACC // NS

_EPS = 1e-5


@functools.cache
def _sc_kernels():
    mesh = plsc.VectorSubcoreMesh(core_axis_name="c", subcore_axis_name="s")

    @functools.partial(
        pl.kernel,
        mesh=mesh,
        compiler_params=pltpu.CompilerParams(needs_layout_passes=False),
        out_type=jax.ShapeDtypeStruct((NW, 1, NPAD), jnp.float32),
        scratch_types=[
            pltpu.VMEM((EPT,), jnp.int32),
            pltpu.VMEM((NPAD,), jnp.float32),
        ],
    )
    def sc_degree(col_hbm, zeros_hbm, out_hbm, colm, dacc):
        c = lax.axis_index("c")
        s = lax.axis_index("s")
        wid = s * NC + c
        pltpu.sync_copy(col_hbm.at[pl.ds(wid * EPT, EPT)], colm)
        pltpu.sync_copy(zeros_hbm, dacc)
        ones16 = jnp.ones((16,), jnp.float32)

        def chunk(t, carry):
            idx = colm[pl.ds(t * 16, 16)]
            plsc.addupdate_scatter(dacc, [idx], ones16)
            return carry

        lax.fori_loop(0, EPT // 16, chunk, 0)
        pltpu.sync_copy(dacc, out_hbm.at[wid, 0])

    @functools.partial(
        pl.kernel,
        mesh=mesh,
        out_type=jax.ShapeDtypeStruct((NC, NACC, D), jnp.float32),
        scratch_types=[
            [pltpu.VMEM((KA,), jnp.int32) for _ in range(NB)],
            [pltpu.VMEM((KA,), jnp.int32) for _ in range(NB)],
            [pltpu.VMEM((KA, D), jnp.float32) for _ in range(NB)],
            pltpu.VMEM_SHARED((NACC, D), jnp.float32),
            [pltpu.SemaphoreType.DMA for _ in range(NB)],
            [pltpu.SemaphoreType.DMA for _ in range(NB)],
            [pltpu.SemaphoreType.DMA for _ in range(NB)],
            [pltpu.SemaphoreType.DMA for _ in range(NB)],
        ],
    )
    def sc_aggregate(h_hbm, row_hbm, col_hbm, zeros_hbm, out_hbm,
                     r, c, buf, acc, g, sc, ir, ic):
        cc_ = lax.axis_index("c")
        ss_ = lax.axis_index("s")
        wid = ss_ * NC + cc_
        pltpu.sync_copy(zeros_hbm, acc.at[pl.ds(ss_ * TPTP, TPTP)])

        base = wid * EPT

        def ldrow(i, b):
            pltpu.async_copy(row_hbm.at[pl.ds(base + i * KA, KA)], r[b], ir[b])

        def ldcol(i, b):
            pltpu.async_copy(col_hbm.at[pl.ds(base + i * KA, KA)], c[b], ic[b])

        def irwait(b):
            pltpu.make_async_copy(row_hbm.at[pl.ds(base, KA)], r[b], ir[b]).wait()

        def icwait(b):
            pltpu.make_async_copy(col_hbm.at[pl.ds(base, KA)], c[b], ic[b]).wait()

        def gather(b):
            pltpu.async_copy(h_hbm.at[r[b]], buf[b], g[b])

        def gwait(b):
            pltpu.make_async_copy(h_hbm.at[r[0]], buf[b], g[b]).wait()

        def scat(b):
            pltpu.async_copy(buf[b], acc.at[c[b]], sc[b], add=True)

        def swait(b):
            pltpu.make_async_copy(buf[b], acc.at[c[0]], sc[b]).wait()

        for b in range(NB):
            ldrow(b, b)
        for b in range(G):
            ldcol(b, b)
        plsc.subcore_barrier()
        for b in range(G):
            irwait(b)
            gather(b)

        def step(i, b):
            b3 = (b + G) % NB
            gwait(b)
            @pl.when(i + NB < NCA)
            def _ldr():
                ldrow(i + NB, b)
            icwait(b)
            scat(b)
            @pl.when(i + G < NCA)
            def _nxt():
                @pl.when(i + G - NB >= 0)
                def _sw():
                    swait(b3)
                ldcol(i + G, b3)
                irwait(b3)
                gather(b3)

        def group(j, carry):
            i0 = NB * j
            for b in range(NB):
                step(i0 + b, b)
            return carry

        lax.fori_loop(0, NCA // NB, group, 0)
        for i in range(NCA - (NCA % NB), NCA):
            step(i, i % NB)
        for i in range(NCA - NB, NCA):
            swait(i % NB)
        plsc.subcore_barrier()
        pltpu.sync_copy(acc.at[pl.ds(ss_ * TPTP, TPTP)],
                        out_hbm.at[cc_, pl.ds(ss_ * TPTP, TPTP)])

    return sc_degree, sc_aggregate


def _tc_dinv_body(deg_ref, out_ref):
    deg = jnp.sum(deg_ref[:, 0, :N], axis=0, keepdims=True) + 1.0
    out_ref[...] = lax.rsqrt(deg).reshape(N, 1)


def _tc_first_body(x_ref, w_ref, dinv_ref, out_ref):
    h = jnp.dot(x_ref[...], w_ref[...], preferred_element_type=jnp.float32)
    out_ref[...] = h * dinv_ref[...]


def _tc_mid_body(s_ref, hp_ref, dinv_ref, b_ref, g_ref, be_ref, w_ref, out_ref):
    dinv = dinv_ref[...]
    conv = (s_ref[0, :N] + s_ref[1, :N] + hp_ref[...]) * dinv + b_ref[...]
    m = jnp.mean(conv, axis=0, keepdims=True)
    cc = conv - m
    v = jnp.mean(cc * cc, axis=0, keepdims=True)
    y = cc * lax.rsqrt(v + _EPS) * g_ref[...] + be_ref[...]
    y = jnp.maximum(y, 0.0)
    out_ref[...] = jnp.dot(y, w_ref[...],
                           preferred_element_type=jnp.float32) * dinv


def _tc_last_body(s_ref, hp_ref, dinv_ref, b_ref, g_ref, be_ref, out_ref):
    conv = (s_ref[0, :N] + s_ref[1, :N] + hp_ref[...]) * dinv_ref[...] + b_ref[...]
    m = jnp.mean(conv, axis=0, keepdims=True)
    cc = conv - m
    v = jnp.mean(cc * cc, axis=0, keepdims=True)
    out_ref[...] = cc * lax.rsqrt(v + _EPS) * g_ref[...] + be_ref[...]


_nd_f32 = jax.ShapeDtypeStruct((N, D), jnp.float32)

_tc_dinv = pl.pallas_call(
    _tc_dinv_body, out_shape=jax.ShapeDtypeStruct((N, 1), jnp.float32))
_tc_first = pl.pallas_call(_tc_first_body, out_shape=_nd_f32)
_tc_mid = pl.pallas_call(_tc_mid_body, out_shape=_nd_f32)
_tc_last = pl.pallas_call(_tc_last_body, out_shape=_nd_f32)


def kernel(x, edge_index, W1, b1, g1, be1, W2, b2, g2, be2, W3, b3, g3, be3):
    sc_degree, sc_aggregate = _sc_kernels()
    row_mat = edge_index[0]
    col_mat = edge_index[1]

    zeros_deg = jnp.zeros((NPAD,), jnp.float32)
    zeros_acc = jnp.zeros((TPTP, D), jnp.float32)

    deg_part = sc_degree(col_mat, zeros_deg)
    dinv = _tc_dinv(deg_part)

    b1r, g1r, be1r = b1.reshape(1, D), g1.reshape(1, D), be1.reshape(1, D)
    b2r, g2r, be2r = b2.reshape(1, D), g2.reshape(1, D), be2.reshape(1, D)
    b3r, g3r, be3r = b3.reshape(1, D), g3.reshape(1, D), be3.reshape(1, D)

    hp = _tc_first(x, W1, dinv)
    S = sc_aggregate(hp, row_mat, col_mat, zeros_acc)
    hp = _tc_mid(S, hp, dinv, b1r, g1r, be1r, W2)
    S = sc_aggregate(hp, row_mat, col_mat, zeros_acc)
    hp = _tc_mid(S, hp, dinv, b2r, g2r, be2r, W3)
    S = sc_aggregate(hp, row_mat, col_mat, zeros_acc)
    return _tc_last(S, hp, dinv, b3r, g3r, be3r)


# speedup vs baseline: 1.1752x; 1.0271x over previous
import functools

import jax
import jax.numpy as jnp
from jax import lax
from jax.experimental import pallas as pl
from jax.experimental.pallas import tpu as pltpu
from jax.experimental.pallas import tpu_sc as plsc

N = 10000
E = 320000
F = 128
D = 128

NC = 2
NS = 16
NW = NC * NS

EPT = E // NW
K = 80
NCHUNK = EPT // K
KA = 40
NCA = EPT // KA

NB = 6
G = 5
NPAD = 10240
DSL = NPAD // NS
DW = 16
NACC = 10240
TPTP = NACC // NS

_EPS = 1e-5


@functools.cache
def _sc_kernels():
    mesh = plsc.VectorSubcoreMesh(core_axis_name="c", subcore_axis_name="s")

    @functools.partial(
        pl.kernel,
        mesh=mesh,
        compiler_params=pltpu.CompilerParams(needs_layout_passes=False),
        out_type=jax.ShapeDtypeStruct((NW, 1, NPAD), jnp.float32),
        scratch_types=[
            pltpu.VMEM((EPT,), jnp.int32),
            pltpu.VMEM((NPAD,), jnp.float32),
        ],
    )
    def sc_degree(ei_hbm, zeros_hbm, out_hbm, colm, dacc):
        c = lax.axis_index("c")
        s = lax.axis_index("s")
        wid = s * NC + c
        pltpu.sync_copy(ei_hbm.at[pl.ds(E + wid * EPT, EPT)], colm)
        pltpu.sync_copy(zeros_hbm, dacc)
        ones16 = jnp.ones((16,), jnp.float32)

        def chunk(t, carry):
            idx = colm[pl.ds(t * 16, 16)]
            plsc.addupdate_scatter(dacc, [idx], ones16)
            return carry

        lax.fori_loop(0, EPT // 16, chunk, 0)
        pltpu.sync_copy(dacc, out_hbm.at[wid, 0])

    @functools.partial(
        pl.kernel,
        mesh=mesh,
        out_type=jax.ShapeDtypeStruct((NC, NACC, D), jnp.float32),
        scratch_types=[
            [pltpu.VMEM((KA,), jnp.int32) for _ in range(NB)],
            [pltpu.VMEM((KA,), jnp.int32) for _ in range(NB)],
            [pltpu.VMEM((KA, D), jnp.float32) for _ in range(NB)],
            pltpu.VMEM_SHARED((NACC, D), jnp.float32),
            [pltpu.SemaphoreType.DMA for _ in range(NB)],
            [pltpu.SemaphoreType.DMA for _ in range(NB)],
            [pltpu.SemaphoreType.DMA for _ in range(NB)],
            [pltpu.SemaphoreType.DMA for _ in range(NB)],
        ],
    )
    def sc_aggregate(h_hbm, ei_hbm, zeros_hbm, out_hbm,
                     r, c, buf, acc, g, sc, ir, ic):
        cc_ = lax.axis_index("c")
        ss_ = lax.axis_index("s")
        wid = ss_ * NC + cc_
        pltpu.sync_copy(zeros_hbm, acc.at[pl.ds(ss_ * TPTP, TPTP)])

        base = wid * EPT

        def ldrow(i, b):
            pltpu.async_copy(ei_hbm.at[pl.ds(base + i * KA, KA)], r[b], ir[b])

        def ldcol(i, b):
            pltpu.async_copy(ei_hbm.at[pl.ds(E + base + i * KA, KA)], c[b], ic[b])

        def irwait(b):
            pltpu.make_async_copy(ei_hbm.at[pl.ds(base, KA)], r[b], ir[b]).wait()

        def icwait(b):
            pltpu.make_async_copy(ei_hbm.at[pl.ds(E + base, KA)], c[b], ic[b]).wait()

        def gather(b):
            pltpu.async_copy(h_hbm.at[r[b]], buf[b], g[b])

        def gwait(b):
            pltpu.make_async_copy(h_hbm.at[r[0]], buf[b], g[b]).wait()

        def scat(b):
            pltpu.async_copy(buf[b], acc.at[c[b]], sc[b], add=True)

        def swait(b):
            pltpu.make_async_copy(buf[b], acc.at[c[0]], sc[b]).wait()

        for b in range(NB):
            ldrow(b, b)
        for b in range(G):
            ldcol(b, b)
        plsc.subcore_barrier()
        for b in range(G):
            irwait(b)
            gather(b)

        def step(i, b):
            b3 = (b + G) % NB
            gwait(b)
            @pl.when(i + NB < NCA)
            def _ldr():
                ldrow(i + NB, b)
            icwait(b)
            scat(b)
            @pl.when(i + G < NCA)
            def _nxt():
                @pl.when(i + G - NB >= 0)
                def _sw():
                    swait(b3)
                ldcol(i + G, b3)
                irwait(b3)
                gather(b3)

        def group(j, carry):
            i0 = NB * j
            for b in range(NB):
                step(i0 + b, b)
            return carry

        lax.fori_loop(0, NCA // NB, group, 0)
        for i in range(NCA - (NCA % NB), NCA):
            step(i, i % NB)
        for i in range(NCA - NB, NCA):
            swait(i % NB)
        plsc.subcore_barrier()
        pltpu.sync_copy(acc.at[pl.ds(ss_ * TPTP, TPTP)],
                        out_hbm.at[cc_, pl.ds(ss_ * TPTP, TPTP)])

    return sc_degree, sc_aggregate


def _tc_dinv_body(deg_ref, out_ref):
    deg = jnp.sum(deg_ref[:, 0, :N], axis=0, keepdims=True) + 1.0
    out_ref[...] = lax.rsqrt(deg).reshape(N, 1)


def _tc_first_body(x_ref, w_ref, dinv_ref, out_ref):
    h = jnp.dot(x_ref[...], w_ref[...], preferred_element_type=jnp.float32)
    out_ref[...] = h * dinv_ref[...]


def _tc_mid_body(s_ref, hp_ref, dinv_ref, b_ref, g_ref, be_ref, w_ref, out_ref):
    dinv = dinv_ref[...]
    conv = (s_ref[0, :N] + s_ref[1, :N] + hp_ref[...]) * dinv + b_ref[...]
    m = jnp.mean(conv, axis=0, keepdims=True)
    cc = conv - m
    v = jnp.mean(cc * cc, axis=0, keepdims=True)
    y = cc * lax.rsqrt(v + _EPS) * g_ref[...] + be_ref[...]
    y = jnp.maximum(y, 0.0)
    out_ref[...] = jnp.dot(y, w_ref[...],
                           preferred_element_type=jnp.float32) * dinv


def _tc_last_body(s_ref, hp_ref, dinv_ref, b_ref, g_ref, be_ref, out_ref):
    conv = (s_ref[0, :N] + s_ref[1, :N] + hp_ref[...]) * dinv_ref[...] + b_ref[...]
    m = jnp.mean(conv, axis=0, keepdims=True)
    cc = conv - m
    v = jnp.mean(cc * cc, axis=0, keepdims=True)
    out_ref[...] = cc * lax.rsqrt(v + _EPS) * g_ref[...] + be_ref[...]


_nd_f32 = jax.ShapeDtypeStruct((N, D), jnp.float32)

_tc_dinv = pl.pallas_call(
    _tc_dinv_body, out_shape=jax.ShapeDtypeStruct((N, 1), jnp.float32))
_tc_first = pl.pallas_call(_tc_first_body, out_shape=_nd_f32)
_tc_mid = pl.pallas_call(_tc_mid_body, out_shape=_nd_f32)
_tc_last = pl.pallas_call(_tc_last_body, out_shape=_nd_f32)


def kernel(x, edge_index, W1, b1, g1, be1, W2, b2, g2, be2, W3, b3, g3, be3):
    sc_degree, sc_aggregate = _sc_kernels()
    ei_flat = edge_index.reshape(2 * E)

    zeros_deg = jnp.zeros((NPAD,), jnp.float32)
    zeros_acc = jnp.zeros((TPTP, D), jnp.float32)

    deg_part = sc_degree(ei_flat, zeros_deg)
    dinv = _tc_dinv(deg_part)

    b1r, g1r, be1r = b1.reshape(1, D), g1.reshape(1, D), be1.reshape(1, D)
    b2r, g2r, be2r = b2.reshape(1, D), g2.reshape(1, D), be2.reshape(1, D)
    b3r, g3r, be3r = b3.reshape(1, D), g3.reshape(1, D), be3.reshape(1, D)

    hp = _tc_first(x, W1, dinv)
    S = sc_aggregate(hp, ei_flat, zeros_acc)
    hp = _tc_mid(S, hp, dinv, b1r, g1r, be1r, W2)
    S = sc_aggregate(hp, ei_flat, zeros_acc)
    hp = _tc_mid(S, hp, dinv, b2r, g2r, be2r, W3)
    S = sc_aggregate(hp, ei_flat, zeros_acc)
    return _tc_last(S, hp, dinv, b3r, g3r, be3r)


# speedup vs baseline: 1.1888x; 1.0116x over previous
import functools

import jax
import jax.numpy as jnp
from jax import lax
from jax.experimental import pallas as pl
from jax.experimental.pallas import tpu as pltpu
from jax.experimental.pallas import tpu_sc as plsc

N = 10000
E = 320000
F = 128
D = 128

NC = 2
NS = 16
NW = NC * NS

EPT = E // NW
K = 80
NCHUNK = EPT // K
KA = 40
NCA = EPT // KA

NB = 6
G = 5
NPAD = 10240
DSL = NPAD // NS
DW = 16
NACC = 10240
TPTP = NACC // NS

_EPS = 1e-5


@functools.cache
def _sc_kernels():
    mesh = plsc.VectorSubcoreMesh(core_axis_name="c", subcore_axis_name="s")

    @functools.partial(
        pl.kernel,
        mesh=mesh,
        compiler_params=pltpu.CompilerParams(needs_layout_passes=False),
        out_type=jax.ShapeDtypeStruct((NW, 1, NPAD), jnp.float32),
        scratch_types=[
            pltpu.VMEM((EPT,), jnp.int32),
            pltpu.VMEM((NPAD,), jnp.float32),
        ],
    )
    def sc_degree(ei_hbm, zeros_hbm, out_hbm, colm, dacc):
        c = lax.axis_index("c")
        s = lax.axis_index("s")
        wid = s * NC + c
        pltpu.sync_copy(ei_hbm.at[pl.ds(E + wid * EPT, EPT)], colm)
        pltpu.sync_copy(zeros_hbm, dacc)
        ones16 = jnp.ones((16,), jnp.float32)

        def chunk(t, carry):
            idx = colm[pl.ds(t * 16, 16)]
            plsc.addupdate_scatter(dacc, [idx], ones16)
            return carry

        lax.fori_loop(0, EPT // 16, chunk, 0)
        pltpu.sync_copy(dacc, out_hbm.at[wid, 0])

    @functools.partial(
        pl.kernel,
        mesh=mesh,
        out_type=jax.ShapeDtypeStruct((NC, NACC, D), jnp.float32),
        scratch_types=[
            [pltpu.VMEM((KA,), jnp.int32) for _ in range(NB)],
            [pltpu.VMEM((KA,), jnp.int32) for _ in range(NB)],
            [pltpu.VMEM((KA, D), jnp.float32) for _ in range(NB)],
            pltpu.VMEM_SHARED((NACC, D), jnp.float32),
            [pltpu.SemaphoreType.DMA for _ in range(NB)],
            [pltpu.SemaphoreType.DMA for _ in range(NB)],
            [pltpu.SemaphoreType.DMA for _ in range(NB)],
            [pltpu.SemaphoreType.DMA for _ in range(NB)],
        ],
    )
    def sc_aggregate(h_hbm, ei_hbm, zeros_hbm, out_hbm,
                     r, c, buf, acc, g, sc, ir, ic):
        cc_ = lax.axis_index("c")
        ss_ = lax.axis_index("s")
        wid = ss_ * NC + cc_
        pltpu.sync_copy(zeros_hbm, acc.at[pl.ds(ss_ * TPTP, TPTP)])

        base = wid * EPT

        def ldrow(i, b):
            pltpu.async_copy(ei_hbm.at[pl.ds(base + i * KA, KA)], r[b], ir[b])

        def ldcol(i, b):
            pltpu.async_copy(ei_hbm.at[pl.ds(E + base + i * KA, KA)], c[b], ic[b])

        def irwait(b):
            pltpu.make_async_copy(ei_hbm.at[pl.ds(base, KA)], r[b], ir[b]).wait()

        def icwait(b):
            pltpu.make_async_copy(ei_hbm.at[pl.ds(E + base, KA)], c[b], ic[b]).wait()

        def gather(b):
            pltpu.async_copy(h_hbm.at[r[b]], buf[b], g[b])

        def gwait(b):
            pltpu.make_async_copy(h_hbm.at[r[0]], buf[b], g[b]).wait()

        def scat(b):
            pltpu.async_copy(buf[b], acc.at[c[b]], sc[b], add=True)

        def swait(b):
            pltpu.make_async_copy(buf[b], acc.at[c[0]], sc[b]).wait()

        for b in range(NB):
            ldrow(b, b)
        for b in range(G):
            ldcol(b, b)
        plsc.subcore_barrier()
        for b in range(G):
            irwait(b)
            gather(b)

        def step(i, b):
            b3 = (b + G) % NB
            gwait(b)
            @pl.when(i + NB < NCA)
            def _ldr():
                ldrow(i + NB, b)
            icwait(b)
            scat(b)
            @pl.when(i + G < NCA)
            def _nxt():
                @pl.when(i + G - NB >= 0)
                def _sw():
                    swait(b3)
                ldcol(i + G, b3)
                irwait(b3)
                gather(b3)

        def group(j, carry):
            i0 = NB * j
            for b in range(NB):
                step(i0 + b, b)
            return carry

        lax.fori_loop(0, NCA // NB, group, 0)
        for i in range(NCA - (NCA % NB), NCA):
            step(i, i % NB)
        for i in range(NCA - NB, NCA):
            swait(i % NB)
        plsc.subcore_barrier()
        pltpu.sync_copy(acc.at[pl.ds(ss_ * TPTP, TPTP)],
                        out_hbm.at[cc_, pl.ds(ss_ * TPTP, TPTP)])

    return sc_degree, sc_aggregate


def _tc_first_body(x_ref, w_ref, deg_ref, out_ref, dinv_ref):
    deg = jnp.sum(deg_ref[:, 0, :N], axis=0, keepdims=True) + 1.0
    dinv = lax.rsqrt(deg).reshape(N, 1)
    dinv_ref[...] = dinv
    h = jnp.dot(x_ref[...], w_ref[...], preferred_element_type=jnp.float32)
    out_ref[...] = h * dinv


def _tc_mid_body(s_ref, hp_ref, dinv_ref, b_ref, g_ref, be_ref, w_ref, out_ref):
    dinv = dinv_ref[...]
    conv = (s_ref[0, :N] + s_ref[1, :N] + hp_ref[...]) * dinv + b_ref[...]
    m = jnp.mean(conv, axis=0, keepdims=True)
    cc = conv - m
    v = jnp.mean(cc * cc, axis=0, keepdims=True)
    y = cc * lax.rsqrt(v + _EPS) * g_ref[...] + be_ref[...]
    y = jnp.maximum(y, 0.0)
    out_ref[...] = jnp.dot(y, w_ref[...],
                           preferred_element_type=jnp.float32) * dinv


def _tc_last_body(s_ref, hp_ref, dinv_ref, b_ref, g_ref, be_ref, out_ref):
    conv = (s_ref[0, :N] + s_ref[1, :N] + hp_ref[...]) * dinv_ref[...] + b_ref[...]
    m = jnp.mean(conv, axis=0, keepdims=True)
    cc = conv - m
    v = jnp.mean(cc * cc, axis=0, keepdims=True)
    out_ref[...] = cc * lax.rsqrt(v + _EPS) * g_ref[...] + be_ref[...]


_nd_f32 = jax.ShapeDtypeStruct((N, D), jnp.float32)

_tc_first = pl.pallas_call(
    _tc_first_body,
    out_shape=[_nd_f32, jax.ShapeDtypeStruct((N, 1), jnp.float32)])
_tc_mid = pl.pallas_call(_tc_mid_body, out_shape=_nd_f32)
_tc_last = pl.pallas_call(_tc_last_body, out_shape=_nd_f32)


def kernel(x, edge_index, W1, b1, g1, be1, W2, b2, g2, be2, W3, b3, g3, be3):
    sc_degree, sc_aggregate = _sc_kernels()
    ei_flat = edge_index.reshape(2 * E)

    zeros_deg = jnp.zeros((NPAD,), jnp.float32)
    zeros_acc = jnp.zeros((TPTP, D), jnp.float32)

    deg_part = sc_degree(ei_flat, zeros_deg)

    b1r, g1r, be1r = b1.reshape(1, D), g1.reshape(1, D), be1.reshape(1, D)
    b2r, g2r, be2r = b2.reshape(1, D), g2.reshape(1, D), be2.reshape(1, D)
    b3r, g3r, be3r = b3.reshape(1, D), g3.reshape(1, D), be3.reshape(1, D)

    hp, dinv = _tc_first(x, W1, deg_part)
    S = sc_aggregate(hp, ei_flat, zeros_acc)
    hp = _tc_mid(S, hp, dinv, b1r, g1r, be1r, W2)
    S = sc_aggregate(hp, ei_flat, zeros_acc)
    hp = _tc_mid(S, hp, dinv, b2r, g2r, be2r, W3)
    S = sc_aggregate(hp, ei_flat, zeros_acc)
    return _tc_last(S, hp, dinv, b3r, g3r, be3r)
